# baseline (speedup 1.0000x reference)
import functools

import jax
import jax.numpy as jnp
from jax import lax
from jax.experimental import pallas as pl
from jax.experimental.pallas import tpu as pltpu
from jax.experimental.pallas import tpu_sc as plsc


def _concat_tables(old2, new2, n_old_blk, n_new_blk, blk):
    def body(old_ref, new_ref, out_ref):
        i = pl.program_id(0)

        @pl.when(i < n_old_blk)
        def _():
            out_ref[...] = old_ref[...]

        @pl.when(i >= n_old_blk)
        def _():
            out_ref[...] = new_ref[...]

    total = n_old_blk + n_new_blk
    return pl.pallas_call(
        body,
        grid=(total,),
        in_specs=[
            pl.BlockSpec((blk, 128), lambda i: (jnp.minimum(i, n_old_blk - 1), 0)),
            pl.BlockSpec((blk, 128), lambda i: (jnp.maximum(i - n_old_blk, 0), 0)),
        ],
        out_specs=pl.BlockSpec((blk, 128), lambda i: (i, 0)),
        out_shape=jax.ShapeDtypeStruct((total * blk, 128), jnp.float32),
    )(old2, new2)


def _transpose_blocks(rows3, nb_total, k0, acc, hist, d):
    nb, bw = rows3.shape[0], rows3.shape[1]
    out_shape = jax.ShapeDtypeStruct((hist, d // 8, nb_total, 8, bw), jnp.float32)

    def body(in_ref, *refs):
        out_ref = refs[-1]
        x = in_ref[...].reshape(bw, hist * d)
        z = jnp.transpose(x)
        out_ref[...] = z.reshape(hist, d // 8, 1, 8, bw)

    in_specs = [pl.BlockSpec((1, bw, hist * d), lambda i: (i, 0, 0))]
    args = (rows3,)
    aliases = {}
    if acc is not None:
        in_specs.append(pl.BlockSpec(memory_space=pl.ANY))
        args = (rows3, acc)
        aliases = {1: 0}
    return pl.pallas_call(
        body,
        grid=(nb,),
        in_specs=in_specs,
        out_specs=pl.BlockSpec(
            (hist, d // 8, 1, 8, bw), lambda i: (0, 0, k0 + i, 0, 0)
        ),
        out_shape=out_shape,
        input_output_aliases=aliases,
    )(*args)


def kernel(input_ids, old_weight, new_weight):
    old_vocab, d = old_weight.shape
    new_vocab = new_weight.shape[0]
    batch, hist = input_ids.shape
    n = batch * hist

    packf = 128 // d
    blk = 1000
    n_old_blk = old_vocab // packf // blk
    n_new_blk = new_vocab // packf // blk
    combined = _concat_tables(
        old_weight.reshape(old_vocab // packf, 128),
        new_weight.reshape(new_vocab // packf, 128),
        n_old_blk,
        n_new_blk,
        blk,
    ).reshape(old_vocab + new_vocab, d)

    info = plsc.get_sparse_core_info()
    nc, ns = info.num_cores, info.num_subcores
    nw = nc * ns
    bw = batch // nw
    sub = 128
    nsl = 8
    n_sl = n // nsl
    per_w = n_sl // nw
    ch = 640
    iters = per_w // ch
    nb_sl = nw // nsl

    mesh = plsc.VectorSubcoreMesh(core_axis_name="c", subcore_axis_name="s")

    def make_gather(slice_base):
        @functools.partial(
            pl.kernel,
            mesh=mesh,
            compiler_params=pltpu.CompilerParams(use_tc_tiling_on_sc=False),
            out_type=jax.ShapeDtypeStruct((n_sl, d), jnp.float32),
            scratch_types=[
                pltpu.VMEM((ch,), jnp.int32),
                pltpu.VMEM((ch, d), jnp.float32),
                pltpu.SemaphoreType.DMA,
            ],
        )
        def gather_k(tbl_hbm, ids_hbm, out_hbm, idx_v, rows_v, sem):
            wid = lax.axis_index("s") * nc + lax.axis_index("c")
            base = wid * per_w

            def body(it, carry):
                off = base + it * ch
                pltpu.sync_copy(ids_hbm.at[pl.ds(slice_base + off, ch)], idx_v)
                copies = []
                for k in range(ch // sub):
                    copies.append(
                        pltpu.async_copy(
                            tbl_hbm.at[idx_v.at[pl.ds(k * sub, sub)]],
                            rows_v.at[pl.ds(k * sub, sub)],
                            sem,
                        )
                    )
                for cp in copies:
                    cp.wait()
                pltpu.sync_copy(rows_v, out_hbm.at[pl.ds(off, ch)])
                return carry

            lax.fori_loop(0, iters, body, 0)

        return gather_k

    ids_bh = input_ids.astype(jnp.int32).reshape(n)
    acc = None
    for k in range(nsl):
        rows_k = make_gather(k * n_sl)(combined, ids_bh)
        rows3 = rows_k.reshape(nb_sl, bw, hist * d)
        acc = _transpose_blocks(rows3, nw, k * nb_sl, acc, hist, d)

    out6 = acc.reshape(hist, d // 8, nw, 8, bw)
    return jnp.transpose(out6, (2, 4, 0, 1, 3)).reshape(batch, hist, d)


# speedup vs baseline: 1.0009x; 1.0009x over previous
import functools

import jax
import jax.numpy as jnp
from jax import lax
from jax.experimental import pallas as pl
from jax.experimental.pallas import tpu as pltpu
from jax.experimental.pallas import tpu_sc as plsc


def _concat_tables(old2, new2, n_old_blk, n_new_blk, blk):
    def body(old_ref, new_ref, out_ref):
        i = pl.program_id(0)

        @pl.when(i < n_old_blk)
        def _():
            out_ref[...] = old_ref[...]

        @pl.when(i >= n_old_blk)
        def _():
            out_ref[...] = new_ref[...]

    total = n_old_blk + n_new_blk
    return pl.pallas_call(
        body,
        grid=(total,),
        in_specs=[
            pl.BlockSpec((blk, 128), lambda i: (jnp.minimum(i, n_old_blk - 1), 0)),
            pl.BlockSpec((blk, 128), lambda i: (jnp.maximum(i - n_old_blk, 0), 0)),
        ],
        out_specs=pl.BlockSpec((blk, 128), lambda i: (i, 0)),
        out_shape=jax.ShapeDtypeStruct((total * blk, 128), jnp.float32),
    )(old2, new2)


def _transpose_blocks(rows3, nb_total, k0, acc, hist, d):
    nb, bw = rows3.shape[0], rows3.shape[1]
    out_shape = jax.ShapeDtypeStruct((hist, d // 8, nb_total, 8, bw), jnp.float32)

    def body(in_ref, *refs):
        out_ref = refs[-1]
        x = in_ref[...].reshape(bw, hist * d)
        z = jnp.transpose(x)
        out_ref[...] = z.reshape(hist, d // 8, 1, 8, bw)

    in_specs = [pl.BlockSpec((1, bw, hist * d), lambda i: (i, 0, 0))]
    args = (rows3,)
    aliases = {}
    if acc is not None:
        in_specs.append(pl.BlockSpec(memory_space=pl.ANY))
        args = (rows3, acc)
        aliases = {1: 0}
    return pl.pallas_call(
        body,
        grid=(nb,),
        in_specs=in_specs,
        out_specs=pl.BlockSpec(
            (hist, d // 8, 1, 8, bw), lambda i: (0, 0, k0 + i, 0, 0)
        ),
        out_shape=out_shape,
        input_output_aliases=aliases,
    )(*args)


def kernel(input_ids, old_weight, new_weight):
    old_vocab, d = old_weight.shape
    new_vocab = new_weight.shape[0]
    batch, hist = input_ids.shape
    n = batch * hist

    packf = 128 // d
    blk = 1000
    n_old_blk = old_vocab // packf // blk
    n_new_blk = new_vocab // packf // blk
    combined = _concat_tables(
        old_weight.reshape(old_vocab // packf, 128),
        new_weight.reshape(new_vocab // packf, 128),
        n_old_blk,
        n_new_blk,
        blk,
    ).reshape(old_vocab + new_vocab, d)

    info = plsc.get_sparse_core_info()
    nc, ns = info.num_cores, info.num_subcores
    nw = nc * ns
    bw = batch // nw
    sub = 128
    nsl = 4
    n_sl = n // nsl
    per_w = n_sl // nw
    ch = 640
    iters = per_w // ch
    nb_sl = nw // nsl

    mesh = plsc.VectorSubcoreMesh(core_axis_name="c", subcore_axis_name="s")

    def make_gather(slice_base):
        @functools.partial(
            pl.kernel,
            mesh=mesh,
            compiler_params=pltpu.CompilerParams(use_tc_tiling_on_sc=False),
            out_type=jax.ShapeDtypeStruct((n_sl, d), jnp.float32),
            scratch_types=[
                pltpu.VMEM((ch,), jnp.int32),
                pltpu.VMEM((ch, d), jnp.float32),
                pltpu.SemaphoreType.DMA,
            ],
        )
        def gather_k(tbl_hbm, ids_hbm, out_hbm, idx_v, rows_v, sem):
            wid = lax.axis_index("s") * nc + lax.axis_index("c")
            base = wid * per_w

            def body(it, carry):
                off = base + it * ch
                pltpu.sync_copy(ids_hbm.at[pl.ds(slice_base + off, ch)], idx_v)
                copies = []
                for k in range(ch // sub):
                    copies.append(
                        pltpu.async_copy(
                            tbl_hbm.at[idx_v.at[pl.ds(k * sub, sub)]],
                            rows_v.at[pl.ds(k * sub, sub)],
                            sem,
                        )
                    )
                for cp in copies:
                    cp.wait()
                pltpu.sync_copy(rows_v, out_hbm.at[pl.ds(off, ch)])
                return carry

            lax.fori_loop(0, iters, body, 0)

        return gather_k

    ids_bh = input_ids.astype(jnp.int32).reshape(n)
    acc = None
    for k in range(nsl):
        rows_k = make_gather(k * n_sl)(combined, ids_bh)
        rows3 = rows_k.reshape(nb_sl, bw, hist * d)
        acc = _transpose_blocks(rows3, nw, k * nb_sl, acc, hist, d)

    out6 = acc.reshape(hist, d // 8, nw, 8, bw)
    return jnp.transpose(out6, (2, 4, 0, 1, 3)).reshape(batch, hist, d)
